# baseline (speedup 1.0000x reference)
import functools
from itertools import combinations

import jax
import jax.numpy as jnp
from jax import lax
from jax.experimental import pallas as pl
from jax.experimental.pallas import tpu as pltpu
from jax.experimental.pallas import tpu_sc as plsc

_NUM_CLASSES = 7
_MAX_SET_SIZE = 2


def _class_members():
    mapping = [()]
    for set_size in range(1, _MAX_SET_SIZE + 1):
        for speakers in combinations(range(_NUM_CLASSES), set_size):
            mapping.append(speakers)
    members = [[] for _ in range(_NUM_CLASSES)]
    for k, speakers in enumerate(mapping):
        for v in speakers:
            members[v].append(k)
    return members, len(mapping)


def kernel(powerset, mapping_matrix):
    B, F, K = powerset.shape
    C = mapping_matrix.shape[1]
    N = B * F
    members, npc = _class_members()
    assert npc == K

    info = plsc.get_sparse_core_info()
    NW = info.num_cores * info.num_subcores
    L = info.num_lanes
    RPW = N // NW
    GROUPS = RPW // L

    x_flat = powerset.reshape(N * K)
    mesh = plsc.VectorSubcoreMesh(core_axis_name="c", subcore_axis_name="s")

    @functools.partial(
        pl.kernel,
        mesh=mesh,
        out_type=jax.ShapeDtypeStruct((N * C,), jnp.float32),
        compiler_params=pltpu.CompilerParams(needs_layout_passes=False),
        scratch_types=[
            pltpu.VMEM((RPW * K,), jnp.float32),
            pltpu.VMEM((RPW * C,), jnp.float32),
        ],
    )
    def _powerset_kernel(x_hbm, out_hbm, xv, ov):
        wid = lax.axis_index("s") * info.num_cores + lax.axis_index("c")
        pltpu.sync_copy(x_hbm.at[pl.ds(wid * (RPW * K), RPW * K)], xv)

        lane = lax.iota(jnp.int32, 16)
        lane_k = lane * K
        lane_c = lane * C

        def body(i, carry):
            in_base = i * (L * K)
            out_base = i * (L * C)
            e = []
            for k in range(K):
                v = plsc.load_gather(xv, [lane_k + (in_base + k)])
                e.append(jnp.exp(v))
            acc = e
            while len(acc) > 1:
                acc = [acc[j] + acc[j + 1] for j in range(0, len(acc) - 1, 2)] \
                      + ([acc[-1]] if len(acc) % 2 else [])
            rinv = 1.0 / acc[0]
            for c in range(C):
                s = e[members[c][0]]
                for m in members[c][1:]:
                    s = s + e[m]
                plsc.store_scatter(ov, [lane_c + (out_base + c)], s * rinv)
            return carry

        lax.fori_loop(0, GROUPS, body, 0)
        pltpu.sync_copy(ov, out_hbm.at[pl.ds(wid * (RPW * C), RPW * C)])

    out = _powerset_kernel(x_flat)
    return out.reshape(B, F, C)
